# baseline (speedup 1.0000x reference)
import functools

import jax
import jax.numpy as jnp
from jax import lax
from jax.experimental import pallas as pl
from jax.experimental.pallas import tpu as pltpu
from jax.experimental.pallas import tpu_sc as plsc

D = 64
SCALE = 8.0
NC, NS = 2, 16
NW = NC * NS
BW = 512
CR = 256
NH = 50
NJ = 2 * NH
NBUF = 3

TL = 1024
NBLK = 489
H = NBLK * TL
NLAST = 976


def _detranspose_table(table_t):
    def body(a_ref, b_ref, o_ref):
        eye = jnp.eye(128, dtype=jnp.float32)
        dn = (((0,), (0,)), ((), ()))
        c = jnp.concatenate([a_ref[...], b_ref[...]], axis=0)
        o_ref[...] = jax.lax.dot_general(
            c, eye, dn, preferred_element_type=jnp.float32
        )

    return pl.pallas_call(
        body,
        grid=(NBLK,),
        in_specs=[
            pl.BlockSpec((64, TL), lambda i: (0, i)),
            pl.BlockSpec((64, TL), lambda i: (0, jnp.minimum(i + NBLK, NLAST))),
        ],
        out_specs=pl.BlockSpec((TL, 128), lambda i: (i, 0)),
        out_shape=jax.ShapeDtypeStruct((H, 128), jnp.float32),
    )(table_t, table_t)


def _untranspose_out(hm):
    def body(x_ref, o_ref):
        eye = jnp.eye(D, dtype=jnp.float32) * SCALE
        t = lax.dot_general(
            eye, x_ref[0], (((1,), (1,)), ((), ())),
            preferred_element_type=jnp.float32,
        )
        r4 = t.reshape(8, 8, 16, 128)
        o_ref[...] = jnp.transpose(r4, (0, 2, 1, 3))

    return pl.pallas_call(
        body,
        grid=(NH, 8),
        in_specs=[pl.BlockSpec((1, 2048, D), lambda h, g: (h, g, 0))],
        out_specs=pl.BlockSpec((8, 16, 8, 128), lambda h, g: (h, g, 0, 0)),
        out_shape=jax.ShapeDtypeStruct((NH * 8, 128, 8, 128), jnp.float32),
    )(hm)


@jax.jit
def _sc_gather(xt, table):
    @functools.partial(
        pl.kernel,
        out_type=jax.ShapeDtypeStruct((NH, 16384, D), jnp.float32),
        mesh=plsc.VectorSubcoreMesh(core_axis_name="c", subcore_axis_name="s"),
        scratch_types=[
            pltpu.VMEM((NH, BW), jnp.int32),
            pltpu.VMEM((1, CR, D), jnp.float32),
            pltpu.VMEM((1, CR, D), jnp.float32),
            pltpu.VMEM((1, CR, D), jnp.float32),
            pltpu.SemaphoreType.DMA,
            pltpu.SemaphoreType.DMA,
            pltpu.SemaphoreType.DMA,
        ],
        compiler_params=pltpu.CompilerParams(use_tc_tiling_on_sc=False),
    )
    def body(xt_hbm, tab_hbm, out_hbm, idx_v, r0, r1, r2, s0, s1, s2):
        wid = lax.axis_index("s") * NC + lax.axis_index("c")
        b0w = BW * wid
        bufs = (r0, r1, r2)
        sems = (s0, s1, s2)

        pltpu.sync_copy(xt_hbm.at[:, pl.ds(b0w, BW)], idx_v)

        def fire(j, b):
            h, p2 = j // 2, j % 2
            for g in range(2):
                pltpu.async_copy(
                    tab_hbm.at[idx_v.at[h, pl.ds(p2 * CR + g * 128, 128)]],
                    bufs[b].at[0, pl.ds(g * 128, 128)],
                    sems[b],
                )

        def drain_gather(j, b):
            h, p2 = j // 2, j % 2
            for g in range(2):
                pltpu.make_async_copy(
                    tab_hbm.at[idx_v.at[h, pl.ds(p2 * CR + g * 128, 128)]],
                    bufs[b].at[0, pl.ds(g * 128, 128)],
                    sems[b],
                ).wait()

        def store(j, b):
            h, p2 = j // 2, j % 2
            pltpu.async_copy(
                bufs[b],
                out_hbm.at[pl.ds(h, 1), pl.ds(b0w + p2 * CR, CR)],
                sems[b],
            )

        def drain_store(j, b):
            h, p2 = j // 2, j % 2
            pltpu.make_async_copy(
                bufs[b],
                out_hbm.at[pl.ds(h, 1), pl.ds(b0w + p2 * CR, CR)],
                sems[b],
            ).wait()

        def block(jj, bmod, fire_next=True, drain_prev=True):
            nb = (bmod + 1) % NBUF
            if drain_prev:
                drain_store(jj - 2, nb)
            if fire_next:
                fire(jj + 1, nb)
            drain_gather(jj, bmod)
            store(jj, bmod)

        fire(0, 0)
        block(0, 0, drain_prev=False)
        block(1, 1, drain_prev=False)
        block(2, 2)
        block(3, 0)

        def triple(t, carry):
            j = 4 + 3 * t
            block(j, 1)
            block(j + 1, 2)
            block(j + 2, 0)
            return carry

        lax.fori_loop(0, (NJ - 6) // 3, triple, None)

        block(NJ - 3, (NJ - 3) % NBUF)
        block(NJ - 2, (NJ - 2) % NBUF)
        block(NJ - 1, (NJ - 1) % NBUF, fire_next=False)
        drain_store(NJ - 2, (NJ - 2) % NBUF)
        drain_store(NJ - 1, (NJ - 1) % NBUF)

    return body(xt, table)


def kernel(x, input_embedding):
    b, h = x.shape
    xi = x.astype(jnp.int32)
    xt = jnp.where(xi < H, xi * 2, (xi - H) * 2 + 1).T
    tab2 = _detranspose_table(input_embedding.T)
    tab_rm = tab2.reshape(2 * H, D)
    hm = _sc_gather(xt, tab_rm)
    out4 = _untranspose_out(hm)
    o5 = out4.reshape(NH, 8, 128, 8, 128)
    return jnp.transpose(o5, (2, 4, 0, 1, 3)).reshape(b, h, D)


# speedup vs baseline: 1.1724x; 1.1724x over previous
import functools

import jax
import jax.numpy as jnp
from jax import lax
from jax.experimental import pallas as pl
from jax.experimental.pallas import tpu as pltpu
from jax.experimental.pallas import tpu_sc as plsc

D = 64
SCALE = 8.0
NC, NS = 2, 16
NW = NC * NS
BW = 512
CR = 256
NH = 50
NJ = 2 * NH
NBUF = 3

TL = 1024
NBLK = 489
H = NBLK * TL
NLAST = 976


def _detranspose_table(table_t):
    def body(a_ref, b_ref, o_ref):
        eye = jnp.eye(128, dtype=jnp.float32)
        dn = (((0,), (0,)), ((), ()))
        c = jnp.concatenate([a_ref[...], b_ref[...]], axis=0)
        o_ref[...] = jax.lax.dot_general(
            c, eye, dn, preferred_element_type=jnp.float32
        )

    return pl.pallas_call(
        body,
        grid=(NBLK,),
        in_specs=[
            pl.BlockSpec((64, TL), lambda i: (0, i)),
            pl.BlockSpec((64, TL), lambda i: (0, jnp.minimum(i + NBLK, NLAST))),
        ],
        out_specs=pl.BlockSpec((TL, 128), lambda i: (i, 0)),
        out_shape=jax.ShapeDtypeStruct((H, 128), jnp.float32),
    )(table_t, table_t)


def _untranspose_out(hm):
    def body(x_ref, o_ref):
        eye = jnp.eye(D, dtype=jnp.float32) * SCALE
        t = lax.dot_general(
            eye, x_ref[0], (((1,), (1,)), ((), ())),
            preferred_element_type=jnp.float32,
        )
        r4 = t.reshape(8, 8, 128, 128)
        o_ref[...] = jnp.transpose(r4, (0, 2, 1, 3))

    return pl.pallas_call(
        body,
        grid=(NH,),
        in_specs=[pl.BlockSpec((1, 16384, D), lambda h: (h, 0, 0))],
        out_specs=pl.BlockSpec((8, 128, 8, 128), lambda h: (h, 0, 0, 0)),
        out_shape=jax.ShapeDtypeStruct((NH * 8, 128, 8, 128), jnp.float32),
    )(hm)


@jax.jit
def _sc_gather(xt, table):
    @functools.partial(
        pl.kernel,
        out_type=jax.ShapeDtypeStruct((NH, 16384, D), jnp.float32),
        mesh=plsc.VectorSubcoreMesh(core_axis_name="c", subcore_axis_name="s"),
        scratch_types=[
            pltpu.VMEM((NH, BW), jnp.int32),
            pltpu.VMEM((1, CR, D), jnp.float32),
            pltpu.VMEM((1, CR, D), jnp.float32),
            pltpu.VMEM((1, CR, D), jnp.float32),
            pltpu.SemaphoreType.DMA,
            pltpu.SemaphoreType.DMA,
            pltpu.SemaphoreType.DMA,
        ],
        compiler_params=pltpu.CompilerParams(use_tc_tiling_on_sc=False),
    )
    def body(xt_hbm, tab_hbm, out_hbm, idx_v, r0, r1, r2, s0, s1, s2):
        wid = lax.axis_index("s") * NC + lax.axis_index("c")
        b0w = BW * wid
        bufs = (r0, r1, r2)
        sems = (s0, s1, s2)

        pltpu.sync_copy(xt_hbm.at[:, pl.ds(b0w, BW)], idx_v)

        def fire(j, b):
            h, p2 = j // 2, j % 2
            for g in range(2):
                pltpu.async_copy(
                    tab_hbm.at[idx_v.at[h, pl.ds(p2 * CR + g * 128, 128)]],
                    bufs[b].at[0, pl.ds(g * 128, 128)],
                    sems[b],
                )

        def drain_gather(j, b):
            h, p2 = j // 2, j % 2
            for g in range(2):
                pltpu.make_async_copy(
                    tab_hbm.at[idx_v.at[h, pl.ds(p2 * CR + g * 128, 128)]],
                    bufs[b].at[0, pl.ds(g * 128, 128)],
                    sems[b],
                ).wait()

        def store(j, b):
            h, p2 = j // 2, j % 2
            pltpu.async_copy(
                bufs[b],
                out_hbm.at[pl.ds(h, 1), pl.ds(b0w + p2 * CR, CR)],
                sems[b],
            )

        def drain_store(j, b):
            h, p2 = j // 2, j % 2
            pltpu.make_async_copy(
                bufs[b],
                out_hbm.at[pl.ds(h, 1), pl.ds(b0w + p2 * CR, CR)],
                sems[b],
            ).wait()

        def block(jj, bmod, fire_next=True, drain_prev=True):
            nb = (bmod + 1) % NBUF
            if drain_prev:
                drain_store(jj - 2, nb)
            if fire_next:
                fire(jj + 1, nb)
            drain_gather(jj, bmod)
            store(jj, bmod)

        fire(0, 0)
        block(0, 0, drain_prev=False)
        block(1, 1, drain_prev=False)
        block(2, 2)
        block(3, 0)

        def triple(t, carry):
            j = 4 + 3 * t
            block(j, 1)
            block(j + 1, 2)
            block(j + 2, 0)
            return carry

        lax.fori_loop(0, (NJ - 6) // 3, triple, None)

        block(NJ - 3, (NJ - 3) % NBUF)
        block(NJ - 2, (NJ - 2) % NBUF)
        block(NJ - 1, (NJ - 1) % NBUF, fire_next=False)
        drain_store(NJ - 2, (NJ - 2) % NBUF)
        drain_store(NJ - 1, (NJ - 1) % NBUF)

    return body(xt, table)


def kernel(x, input_embedding):
    b, h = x.shape
    xi = x.astype(jnp.int32)
    xt = jnp.where(xi < H, xi * 2, (xi - H) * 2 + 1).T
    tab2 = _detranspose_table(input_embedding.T)
    tab_rm = tab2.reshape(2 * H, D)
    hm = _sc_gather(xt, tab_rm)
    out4 = _untranspose_out(hm)
    o5 = out4.reshape(NH, 8, 128, 8, 128)
    return jnp.transpose(o5, (2, 4, 0, 1, 3)).reshape(b, h, D)


# speedup vs baseline: 1.3361x; 1.1396x over previous
import functools

import jax
import jax.numpy as jnp
from jax import lax
from jax.experimental import pallas as pl
from jax.experimental.pallas import tpu as pltpu
from jax.experimental.pallas import tpu_sc as plsc

D = 64
SCALE = 8.0
NC, NS = 2, 16
NW = NC * NS
BW = 512
CR = 256
NH = 50
NJ = 2 * NH
NBUF = 3

TL = 2048
NBLK = 245
H = NBLK * TL
NLAST = 488


def _detranspose_table(table_t):
    def body(a_ref, b_ref, o_ref):
        eye = jnp.eye(128, dtype=jnp.float32)
        dn = (((0,), (0,)), ((), ()))
        c = jnp.concatenate([a_ref[...], b_ref[...]], axis=0)
        o_ref[...] = jax.lax.dot_general(
            c, eye, dn, preferred_element_type=jnp.float32
        )

    return pl.pallas_call(
        body,
        grid=(NBLK,),
        in_specs=[
            pl.BlockSpec((64, TL), lambda i: (0, i)),
            pl.BlockSpec((64, TL), lambda i: (0, jnp.minimum(i + NBLK, NLAST))),
        ],
        out_specs=pl.BlockSpec((TL, 128), lambda i: (i, 0)),
        out_shape=jax.ShapeDtypeStruct((H, 128), jnp.float32),
    )(table_t, table_t)


def _untranspose_out(hm):
    def body(x_ref, o_ref):
        eye = jnp.eye(D, dtype=jnp.float32) * SCALE
        t = lax.dot_general(
            eye, x_ref[0], (((1,), (1,)), ((), ())),
            preferred_element_type=jnp.float32,
        )
        r4 = t.reshape(8, 8, 128, 128)
        o_ref[...] = jnp.transpose(r4, (0, 2, 1, 3))

    return pl.pallas_call(
        body,
        grid=(NH,),
        in_specs=[pl.BlockSpec((1, 16384, D), lambda h: (h, 0, 0))],
        out_specs=pl.BlockSpec((8, 128, 8, 128), lambda h: (h, 0, 0, 0)),
        out_shape=jax.ShapeDtypeStruct((NH * 8, 128, 8, 128), jnp.float32),
    )(hm)


@jax.jit
def _sc_gather(xt, table):
    @functools.partial(
        pl.kernel,
        out_type=jax.ShapeDtypeStruct((NH, 16384, D), jnp.float32),
        mesh=plsc.VectorSubcoreMesh(core_axis_name="c", subcore_axis_name="s"),
        scratch_types=[
            pltpu.VMEM((NH, BW), jnp.int32),
            pltpu.VMEM((1, CR, D), jnp.float32),
            pltpu.VMEM((1, CR, D), jnp.float32),
            pltpu.VMEM((1, CR, D), jnp.float32),
            pltpu.SemaphoreType.DMA,
            pltpu.SemaphoreType.DMA,
            pltpu.SemaphoreType.DMA,
        ],
        compiler_params=pltpu.CompilerParams(use_tc_tiling_on_sc=False),
    )
    def body(xt_hbm, tab_hbm, out_hbm, idx_v, r0, r1, r2, s0, s1, s2):
        wid = lax.axis_index("s") * NC + lax.axis_index("c")
        b0w = BW * wid
        bufs = (r0, r1, r2)
        sems = (s0, s1, s2)

        pltpu.sync_copy(xt_hbm.at[:, pl.ds(b0w, BW)], idx_v)

        def fire(j, b):
            h, p2 = j // 2, j % 2
            for g in range(2):
                pltpu.async_copy(
                    tab_hbm.at[idx_v.at[h, pl.ds(p2 * CR + g * 128, 128)]],
                    bufs[b].at[0, pl.ds(g * 128, 128)],
                    sems[b],
                )

        def drain_gather(j, b):
            h, p2 = j // 2, j % 2
            for g in range(2):
                pltpu.make_async_copy(
                    tab_hbm.at[idx_v.at[h, pl.ds(p2 * CR + g * 128, 128)]],
                    bufs[b].at[0, pl.ds(g * 128, 128)],
                    sems[b],
                ).wait()

        def store(j, b):
            h, p2 = j // 2, j % 2
            pltpu.async_copy(
                bufs[b],
                out_hbm.at[pl.ds(h, 1), pl.ds(b0w + p2 * CR, CR)],
                sems[b],
            )

        def drain_store(j, b):
            h, p2 = j // 2, j % 2
            pltpu.make_async_copy(
                bufs[b],
                out_hbm.at[pl.ds(h, 1), pl.ds(b0w + p2 * CR, CR)],
                sems[b],
            ).wait()

        def block(jj, bmod, fire_next=True, drain_prev=True):
            nb = (bmod + 1) % NBUF
            if drain_prev:
                drain_store(jj - 2, nb)
            if fire_next:
                fire(jj + 1, nb)
            drain_gather(jj, bmod)
            store(jj, bmod)

        fire(0, 0)
        block(0, 0, drain_prev=False)
        block(1, 1, drain_prev=False)
        block(2, 2)
        block(3, 0)

        def triple(t, carry):
            j = 4 + 3 * t
            block(j, 1)
            block(j + 1, 2)
            block(j + 2, 0)
            return carry

        lax.fori_loop(0, (NJ - 6) // 3, triple, None)

        block(NJ - 3, (NJ - 3) % NBUF)
        block(NJ - 2, (NJ - 2) % NBUF)
        block(NJ - 1, (NJ - 1) % NBUF, fire_next=False)
        drain_store(NJ - 2, (NJ - 2) % NBUF)
        drain_store(NJ - 1, (NJ - 1) % NBUF)

    return body(xt, table)


def kernel(x, input_embedding):
    b, h = x.shape
    xi = x.astype(jnp.int32)
    xt = jnp.where(xi < H, xi * 2, (xi - H) * 2 + 1).T
    tab2 = _detranspose_table(input_embedding.T)
    tab_rm = tab2.reshape(2 * H, D)
    hm = _sc_gather(xt, tab_rm)
    out4 = _untranspose_out(hm)
    o5 = out4.reshape(NH, 8, 128, 8, 128)
    return jnp.transpose(o5, (2, 4, 0, 1, 3)).reshape(b, h, D)


# speedup vs baseline: 1.3378x; 1.0013x over previous
import functools

import jax
import jax.numpy as jnp
from jax import lax
from jax.experimental import pallas as pl
from jax.experimental.pallas import tpu as pltpu
from jax.experimental.pallas import tpu_sc as plsc

D = 64
SCALE = 8.0
NC, NS = 2, 16
NW = NC * NS
BW = 512
CR = 256
NH = 50
NJ = 2 * NH
NBUF = 3

TL = 2048
NBLK = 245
H = NBLK * TL
NLAST = 488


def _detranspose_table(table_t):
    def body(a_ref, b_ref, o_ref):
        eye = jnp.eye(128, dtype=jnp.float32)
        dn = (((0,), (0,)), ((), ()))
        c = jnp.concatenate([a_ref[...], b_ref[...]], axis=0)
        o_ref[...] = jax.lax.dot_general(
            c, eye, dn, preferred_element_type=jnp.float32
        )

    return pl.pallas_call(
        body,
        grid=(NBLK,),
        in_specs=[
            pl.BlockSpec((64, TL), lambda i: (0, i)),
            pl.BlockSpec((64, TL), lambda i: (0, jnp.minimum(i + NBLK, NLAST))),
        ],
        out_specs=pl.BlockSpec((TL, 128), lambda i: (i, 0)),
        out_shape=jax.ShapeDtypeStruct((H, 128), jnp.float32),
    )(table_t, table_t)


def _untranspose_out(hm):
    def body(x_ref, o_ref):
        eye = jnp.eye(D, dtype=jnp.float32) * SCALE
        dn = (((1,), (1,)), ((), ()))
        for i in range(2):
            t = lax.dot_general(
                eye, x_ref[i], dn, preferred_element_type=jnp.float32
            )
            r4 = t.reshape(8, 8, 128, 128)
            o_ref[pl.ds(i * 8, 8)] = jnp.transpose(r4, (0, 2, 1, 3))

    return pl.pallas_call(
        body,
        grid=(NH // 2,),
        in_specs=[pl.BlockSpec((2, 16384, D), lambda h: (h, 0, 0))],
        out_specs=pl.BlockSpec((16, 128, 8, 128), lambda h: (h, 0, 0, 0)),
        out_shape=jax.ShapeDtypeStruct((NH * 8, 128, 8, 128), jnp.float32),
    )(hm)


@jax.jit
def _sc_gather(xt, table):
    @functools.partial(
        pl.kernel,
        out_type=jax.ShapeDtypeStruct((NH, 16384, D), jnp.float32),
        mesh=plsc.VectorSubcoreMesh(core_axis_name="c", subcore_axis_name="s"),
        scratch_types=[
            pltpu.VMEM((NH, BW), jnp.int32),
            pltpu.VMEM((1, CR, D), jnp.float32),
            pltpu.VMEM((1, CR, D), jnp.float32),
            pltpu.VMEM((1, CR, D), jnp.float32),
            pltpu.SemaphoreType.DMA,
            pltpu.SemaphoreType.DMA,
            pltpu.SemaphoreType.DMA,
        ],
        compiler_params=pltpu.CompilerParams(use_tc_tiling_on_sc=False),
    )
    def body(xt_hbm, tab_hbm, out_hbm, idx_v, r0, r1, r2, s0, s1, s2):
        wid = lax.axis_index("s") * NC + lax.axis_index("c")
        b0w = BW * wid
        bufs = (r0, r1, r2)
        sems = (s0, s1, s2)

        pltpu.sync_copy(xt_hbm.at[:, pl.ds(b0w, BW)], idx_v)

        def fire(j, b):
            h, p2 = j // 2, j % 2
            for g in range(2):
                pltpu.async_copy(
                    tab_hbm.at[idx_v.at[h, pl.ds(p2 * CR + g * 128, 128)]],
                    bufs[b].at[0, pl.ds(g * 128, 128)],
                    sems[b],
                )

        def drain_gather(j, b):
            h, p2 = j // 2, j % 2
            for g in range(2):
                pltpu.make_async_copy(
                    tab_hbm.at[idx_v.at[h, pl.ds(p2 * CR + g * 128, 128)]],
                    bufs[b].at[0, pl.ds(g * 128, 128)],
                    sems[b],
                ).wait()

        def store(j, b):
            h, p2 = j // 2, j % 2
            pltpu.async_copy(
                bufs[b],
                out_hbm.at[pl.ds(h, 1), pl.ds(b0w + p2 * CR, CR)],
                sems[b],
            )

        def drain_store(j, b):
            h, p2 = j // 2, j % 2
            pltpu.make_async_copy(
                bufs[b],
                out_hbm.at[pl.ds(h, 1), pl.ds(b0w + p2 * CR, CR)],
                sems[b],
            ).wait()

        def block(jj, bmod, fire_next=True, drain_prev=True):
            nb = (bmod + 1) % NBUF
            if drain_prev:
                drain_store(jj - 2, nb)
            if fire_next:
                fire(jj + 1, nb)
            drain_gather(jj, bmod)
            store(jj, bmod)

        fire(0, 0)
        block(0, 0, drain_prev=False)
        block(1, 1, drain_prev=False)
        block(2, 2)
        block(3, 0)

        def triple(t, carry):
            j = 4 + 3 * t
            block(j, 1)
            block(j + 1, 2)
            block(j + 2, 0)
            return carry

        lax.fori_loop(0, (NJ - 6) // 3, triple, None)

        block(NJ - 3, (NJ - 3) % NBUF)
        block(NJ - 2, (NJ - 2) % NBUF)
        block(NJ - 1, (NJ - 1) % NBUF, fire_next=False)
        drain_store(NJ - 2, (NJ - 2) % NBUF)
        drain_store(NJ - 1, (NJ - 1) % NBUF)

    return body(xt, table)


def kernel(x, input_embedding):
    b, h = x.shape
    xi = x.astype(jnp.int32)
    xt = jnp.where(xi < H, xi * 2, (xi - H) * 2 + 1).T
    tab2 = _detranspose_table(input_embedding.T)
    tab_rm = tab2.reshape(2 * H, D)
    hm = _sc_gather(xt, tab_rm)
    out4 = _untranspose_out(hm)
    o5 = out4.reshape(NH, 8, 128, 8, 128)
    return jnp.transpose(o5, (2, 4, 0, 1, 3)).reshape(b, h, D)


# speedup vs baseline: 1.4606x; 1.0918x over previous
import functools

import jax
import jax.numpy as jnp
from jax import lax
from jax.experimental import pallas as pl
from jax.experimental.pallas import tpu as pltpu
from jax.experimental.pallas import tpu_sc as plsc

D = 64
SCALE = 8.0
NC, NS = 2, 16
NW = NC * NS
BW = 512
CR = 256
NH = 50
NJ = 2 * NH
NBUF = 3

TL = 4096
NBLK = 123
H = NBLK * TL
NLAST = 244


def _detranspose_table(table_t):
    def body(a_ref, b_ref, o_ref):
        eye = jnp.eye(128, dtype=jnp.float32)
        dn = (((0,), (0,)), ((), ()))
        c = jnp.concatenate([a_ref[...], b_ref[...]], axis=0)
        o_ref[...] = jax.lax.dot_general(
            c, eye, dn, preferred_element_type=jnp.float32
        )

    return pl.pallas_call(
        body,
        grid=(NBLK,),
        in_specs=[
            pl.BlockSpec((64, TL), lambda i: (0, i)),
            pl.BlockSpec((64, TL), lambda i: (0, jnp.minimum(i + NBLK, NLAST))),
        ],
        out_specs=pl.BlockSpec((TL, 128), lambda i: (i, 0)),
        out_shape=jax.ShapeDtypeStruct((H, 128), jnp.float32),
    )(table_t, table_t)


def _untranspose_out(hm):
    def body(x_ref, o_ref):
        eye = jnp.eye(D, dtype=jnp.float32) * SCALE
        dn = (((1,), (1,)), ((), ()))
        for i in range(2):
            t = lax.dot_general(
                eye, x_ref[i], dn, preferred_element_type=jnp.float32
            )
            r4 = t.reshape(8, 8, 128, 128)
            o_ref[pl.ds(i * 8, 8)] = jnp.transpose(r4, (0, 2, 1, 3))

    return pl.pallas_call(
        body,
        grid=(NH // 2,),
        in_specs=[pl.BlockSpec((2, 16384, D), lambda h: (h, 0, 0))],
        out_specs=pl.BlockSpec((16, 128, 8, 128), lambda h: (h, 0, 0, 0)),
        out_shape=jax.ShapeDtypeStruct((NH * 8, 128, 8, 128), jnp.float32),
    )(hm)


@jax.jit
def _sc_gather(xt, table):
    @functools.partial(
        pl.kernel,
        out_type=jax.ShapeDtypeStruct((NH, 16384, D), jnp.float32),
        mesh=plsc.VectorSubcoreMesh(core_axis_name="c", subcore_axis_name="s"),
        scratch_types=[
            pltpu.VMEM((NH, BW), jnp.int32),
            pltpu.VMEM((1, CR, D), jnp.float32),
            pltpu.VMEM((1, CR, D), jnp.float32),
            pltpu.VMEM((1, CR, D), jnp.float32),
            pltpu.SemaphoreType.DMA,
            pltpu.SemaphoreType.DMA,
            pltpu.SemaphoreType.DMA,
        ],
        compiler_params=pltpu.CompilerParams(use_tc_tiling_on_sc=False),
    )
    def body(xt_hbm, tab_hbm, out_hbm, idx_v, r0, r1, r2, s0, s1, s2):
        wid = lax.axis_index("s") * NC + lax.axis_index("c")
        b0w = BW * wid
        bufs = (r0, r1, r2)
        sems = (s0, s1, s2)

        pltpu.sync_copy(xt_hbm.at[:, pl.ds(b0w, BW)], idx_v)

        def fire(j, b):
            h, p2 = j // 2, j % 2
            for g in range(2):
                pltpu.async_copy(
                    tab_hbm.at[idx_v.at[h, pl.ds(p2 * CR + g * 128, 128)]],
                    bufs[b].at[0, pl.ds(g * 128, 128)],
                    sems[b],
                )

        def drain_gather(j, b):
            h, p2 = j // 2, j % 2
            for g in range(2):
                pltpu.make_async_copy(
                    tab_hbm.at[idx_v.at[h, pl.ds(p2 * CR + g * 128, 128)]],
                    bufs[b].at[0, pl.ds(g * 128, 128)],
                    sems[b],
                ).wait()

        def store(j, b):
            h, p2 = j // 2, j % 2
            pltpu.async_copy(
                bufs[b],
                out_hbm.at[pl.ds(h, 1), pl.ds(b0w + p2 * CR, CR)],
                sems[b],
            )

        def drain_store(j, b):
            h, p2 = j // 2, j % 2
            pltpu.make_async_copy(
                bufs[b],
                out_hbm.at[pl.ds(h, 1), pl.ds(b0w + p2 * CR, CR)],
                sems[b],
            ).wait()

        def block(jj, bmod, fire_next=True, drain_prev=True):
            nb = (bmod + 1) % NBUF
            if drain_prev:
                drain_store(jj - 2, nb)
            if fire_next:
                fire(jj + 1, nb)
            drain_gather(jj, bmod)
            store(jj, bmod)

        fire(0, 0)
        block(0, 0, drain_prev=False)
        block(1, 1, drain_prev=False)
        block(2, 2)
        block(3, 0)

        def triple(t, carry):
            j = 4 + 3 * t
            block(j, 1)
            block(j + 1, 2)
            block(j + 2, 0)
            return carry

        lax.fori_loop(0, (NJ - 6) // 3, triple, None)

        block(NJ - 3, (NJ - 3) % NBUF)
        block(NJ - 2, (NJ - 2) % NBUF)
        block(NJ - 1, (NJ - 1) % NBUF, fire_next=False)
        drain_store(NJ - 2, (NJ - 2) % NBUF)
        drain_store(NJ - 1, (NJ - 1) % NBUF)

    return body(xt, table)


def kernel(x, input_embedding):
    b, h = x.shape
    xi = x.astype(jnp.int32)
    xt = jnp.where(xi < H, xi * 2, (xi - H) * 2 + 1).T
    tab2 = _detranspose_table(input_embedding.T)
    tab_rm = tab2.reshape(2 * H, D)
    hm = _sc_gather(xt, tab_rm)
    out4 = _untranspose_out(hm)
    o5 = out4.reshape(NH, 8, 128, 8, 128)
    return jnp.transpose(o5, (2, 4, 0, 1, 3)).reshape(b, h, D)


# speedup vs baseline: 1.5071x; 1.0319x over previous
import functools

import jax
import jax.numpy as jnp
from jax import lax
from jax.experimental import pallas as pl
from jax.experimental.pallas import tpu as pltpu
from jax.experimental.pallas import tpu_sc as plsc

D = 64
SCALE = 8.0
NC, NS = 2, 16
NW = NC * NS
BW = 512
CR = 256
NH = 50
NJ = 2 * NH
NBUF = 3

TL = 8192
NBLK = 62
H = NBLK * TL
NLAST = 122


def _detranspose_table(table_t):
    def body(a_ref, b_ref, o_ref):
        eye = jnp.eye(128, dtype=jnp.float32)
        dn = (((0,), (0,)), ((), ()))
        c = jnp.concatenate([a_ref[...], b_ref[...]], axis=0)
        o_ref[...] = jax.lax.dot_general(
            c, eye, dn, preferred_element_type=jnp.float32
        )

    return pl.pallas_call(
        body,
        grid=(NBLK,),
        in_specs=[
            pl.BlockSpec((64, TL), lambda i: (0, i)),
            pl.BlockSpec((64, TL), lambda i: (0, jnp.minimum(i + NBLK, NLAST))),
        ],
        out_specs=pl.BlockSpec((TL, 128), lambda i: (i, 0)),
        out_shape=jax.ShapeDtypeStruct((H, 128), jnp.float32),
    )(table_t, table_t)


def _untranspose_out(hm):
    def body(x_ref, o_ref):
        eye = jnp.eye(D, dtype=jnp.float32) * SCALE
        dn = (((1,), (1,)), ((), ()))
        for i in range(2):
            t = lax.dot_general(
                eye, x_ref[i], dn, preferred_element_type=jnp.float32
            )
            r4 = t.reshape(8, 8, 128, 128)
            o_ref[pl.ds(i * 8, 8)] = jnp.transpose(r4, (0, 2, 1, 3))

    return pl.pallas_call(
        body,
        grid=(NH // 2,),
        in_specs=[pl.BlockSpec((2, 16384, D), lambda h: (h, 0, 0))],
        out_specs=pl.BlockSpec((16, 128, 8, 128), lambda h: (h, 0, 0, 0)),
        out_shape=jax.ShapeDtypeStruct((NH * 8, 128, 8, 128), jnp.float32),
    )(hm)


@jax.jit
def _sc_gather(xt, table):
    @functools.partial(
        pl.kernel,
        out_type=jax.ShapeDtypeStruct((NH, 16384, D), jnp.float32),
        mesh=plsc.VectorSubcoreMesh(core_axis_name="c", subcore_axis_name="s"),
        scratch_types=[
            pltpu.VMEM((NH, BW), jnp.int32),
            pltpu.VMEM((1, CR, D), jnp.float32),
            pltpu.VMEM((1, CR, D), jnp.float32),
            pltpu.VMEM((1, CR, D), jnp.float32),
            pltpu.SemaphoreType.DMA,
            pltpu.SemaphoreType.DMA,
            pltpu.SemaphoreType.DMA,
        ],
        compiler_params=pltpu.CompilerParams(use_tc_tiling_on_sc=False),
    )
    def body(xt_hbm, tab_hbm, out_hbm, idx_v, r0, r1, r2, s0, s1, s2):
        wid = lax.axis_index("s") * NC + lax.axis_index("c")
        b0w = BW * wid
        bufs = (r0, r1, r2)
        sems = (s0, s1, s2)

        pltpu.sync_copy(xt_hbm.at[:, pl.ds(b0w, BW)], idx_v)

        def fire(j, b):
            h, p2 = j // 2, j % 2
            for g in range(2):
                pltpu.async_copy(
                    tab_hbm.at[idx_v.at[h, pl.ds(p2 * CR + g * 128, 128)]],
                    bufs[b].at[0, pl.ds(g * 128, 128)],
                    sems[b],
                )

        def drain_gather(j, b):
            h, p2 = j // 2, j % 2
            for g in range(2):
                pltpu.make_async_copy(
                    tab_hbm.at[idx_v.at[h, pl.ds(p2 * CR + g * 128, 128)]],
                    bufs[b].at[0, pl.ds(g * 128, 128)],
                    sems[b],
                ).wait()

        def store(j, b):
            h, p2 = j // 2, j % 2
            pltpu.async_copy(
                bufs[b],
                out_hbm.at[pl.ds(h, 1), pl.ds(b0w + p2 * CR, CR)],
                sems[b],
            )

        def drain_store(j, b):
            h, p2 = j // 2, j % 2
            pltpu.make_async_copy(
                bufs[b],
                out_hbm.at[pl.ds(h, 1), pl.ds(b0w + p2 * CR, CR)],
                sems[b],
            ).wait()

        def block(jj, bmod, fire_next=True, drain_prev=True):
            nb = (bmod + 1) % NBUF
            if drain_prev:
                drain_store(jj - 2, nb)
            if fire_next:
                fire(jj + 1, nb)
            drain_gather(jj, bmod)
            store(jj, bmod)

        fire(0, 0)
        block(0, 0, drain_prev=False)
        block(1, 1, drain_prev=False)
        block(2, 2)
        block(3, 0)

        def triple(t, carry):
            j = 4 + 3 * t
            block(j, 1)
            block(j + 1, 2)
            block(j + 2, 0)
            return carry

        lax.fori_loop(0, (NJ - 6) // 3, triple, None)

        block(NJ - 3, (NJ - 3) % NBUF)
        block(NJ - 2, (NJ - 2) % NBUF)
        block(NJ - 1, (NJ - 1) % NBUF, fire_next=False)
        drain_store(NJ - 2, (NJ - 2) % NBUF)
        drain_store(NJ - 1, (NJ - 1) % NBUF)

    return body(xt, table)


def kernel(x, input_embedding):
    b, h = x.shape
    xi = x.astype(jnp.int32)
    xt = jnp.where(xi < H, xi * 2, (xi - H) * 2 + 1).T
    tab2 = _detranspose_table(input_embedding.T)
    tab_rm = tab2.reshape(2 * H, D)
    hm = _sc_gather(xt, tab_rm)
    out4 = _untranspose_out(hm)
    o5 = out4.reshape(NH, 8, 128, 8, 128)
    return jnp.transpose(o5, (2, 4, 0, 1, 3)).reshape(b, h, D)


# speedup vs baseline: 1.5156x; 1.0056x over previous
import functools

import jax
import jax.numpy as jnp
from jax import lax
from jax.experimental import pallas as pl
from jax.experimental.pallas import tpu as pltpu
from jax.experimental.pallas import tpu_sc as plsc

D = 64
SCALE = 8.0
NC, NS = 2, 16
NW = NC * NS
BW = 512
CR = 256
NH = 50
NJ = 2 * NH
NBUF = 3

TL = 16384
NBLK = 31
H = NBLK * TL
NLAST = 61


def _detranspose_table(table_t):
    def body(a_ref, b_ref, o_ref):
        eye = jnp.eye(128, dtype=jnp.float32)
        dn = (((0,), (0,)), ((), ()))
        c = jnp.concatenate([a_ref[...], b_ref[...]], axis=0)
        o_ref[...] = jax.lax.dot_general(
            c, eye, dn, preferred_element_type=jnp.float32
        )

    return pl.pallas_call(
        body,
        grid=(NBLK,),
        in_specs=[
            pl.BlockSpec((64, TL), lambda i: (0, i)),
            pl.BlockSpec((64, TL), lambda i: (0, jnp.minimum(i + NBLK, NLAST))),
        ],
        out_specs=pl.BlockSpec((TL, 128), lambda i: (i, 0)),
        out_shape=jax.ShapeDtypeStruct((H, 128), jnp.float32),
    )(table_t, table_t)


def _untranspose_out(hm):
    def body(x_ref, o_ref):
        eye = jnp.eye(D, dtype=jnp.float32) * SCALE
        dn = (((1,), (1,)), ((), ()))
        for i in range(2):
            t = lax.dot_general(
                eye, x_ref[i], dn, preferred_element_type=jnp.float32
            )
            r4 = t.reshape(8, 8, 128, 128)
            o_ref[pl.ds(i * 8, 8)] = jnp.transpose(r4, (0, 2, 1, 3))

    return pl.pallas_call(
        body,
        grid=(NH // 2,),
        in_specs=[pl.BlockSpec((2, 16384, D), lambda h: (h, 0, 0))],
        out_specs=pl.BlockSpec((16, 128, 8, 128), lambda h: (h, 0, 0, 0)),
        out_shape=jax.ShapeDtypeStruct((NH * 8, 128, 8, 128), jnp.float32),
    )(hm)


@jax.jit
def _sc_gather(xt, table):
    @functools.partial(
        pl.kernel,
        out_type=jax.ShapeDtypeStruct((NH, 16384, D), jnp.float32),
        mesh=plsc.VectorSubcoreMesh(core_axis_name="c", subcore_axis_name="s"),
        scratch_types=[
            pltpu.VMEM((NH, BW), jnp.int32),
            pltpu.VMEM((1, CR, D), jnp.float32),
            pltpu.VMEM((1, CR, D), jnp.float32),
            pltpu.VMEM((1, CR, D), jnp.float32),
            pltpu.SemaphoreType.DMA,
            pltpu.SemaphoreType.DMA,
            pltpu.SemaphoreType.DMA,
        ],
        compiler_params=pltpu.CompilerParams(use_tc_tiling_on_sc=False),
    )
    def body(xt_hbm, tab_hbm, out_hbm, idx_v, r0, r1, r2, s0, s1, s2):
        wid = lax.axis_index("s") * NC + lax.axis_index("c")
        b0w = BW * wid
        bufs = (r0, r1, r2)
        sems = (s0, s1, s2)

        pltpu.sync_copy(xt_hbm.at[:, pl.ds(b0w, BW)], idx_v)

        def fire(j, b):
            h, p2 = j // 2, j % 2
            for g in range(2):
                pltpu.async_copy(
                    tab_hbm.at[idx_v.at[h, pl.ds(p2 * CR + g * 128, 128)]],
                    bufs[b].at[0, pl.ds(g * 128, 128)],
                    sems[b],
                )

        def drain_gather(j, b):
            h, p2 = j // 2, j % 2
            for g in range(2):
                pltpu.make_async_copy(
                    tab_hbm.at[idx_v.at[h, pl.ds(p2 * CR + g * 128, 128)]],
                    bufs[b].at[0, pl.ds(g * 128, 128)],
                    sems[b],
                ).wait()

        def store(j, b):
            h, p2 = j // 2, j % 2
            pltpu.async_copy(
                bufs[b],
                out_hbm.at[pl.ds(h, 1), pl.ds(b0w + p2 * CR, CR)],
                sems[b],
            )

        def drain_store(j, b):
            h, p2 = j // 2, j % 2
            pltpu.make_async_copy(
                bufs[b],
                out_hbm.at[pl.ds(h, 1), pl.ds(b0w + p2 * CR, CR)],
                sems[b],
            ).wait()

        def block(jj, bmod, fire_next=True, drain_prev=True):
            nb = (bmod + 1) % NBUF
            if drain_prev:
                drain_store(jj - 2, nb)
            if fire_next:
                fire(jj + 1, nb)
            drain_gather(jj, bmod)
            store(jj, bmod)

        fire(0, 0)
        block(0, 0, drain_prev=False)
        block(1, 1, drain_prev=False)
        block(2, 2)
        block(3, 0)

        def triple(t, carry):
            j = 4 + 3 * t
            block(j, 1)
            block(j + 1, 2)
            block(j + 2, 0)
            return carry

        lax.fori_loop(0, (NJ - 6) // 3, triple, None)

        block(NJ - 3, (NJ - 3) % NBUF)
        block(NJ - 2, (NJ - 2) % NBUF)
        block(NJ - 1, (NJ - 1) % NBUF, fire_next=False)
        drain_store(NJ - 2, (NJ - 2) % NBUF)
        drain_store(NJ - 1, (NJ - 1) % NBUF)

    return body(xt, table)


def kernel(x, input_embedding):
    b, h = x.shape
    xi = x.astype(jnp.int32)
    xt = jnp.where(xi < H, xi * 2, (xi - H) * 2 + 1).T
    tab2 = _detranspose_table(input_embedding.T)
    tab_rm = tab2.reshape(2 * H, D)
    hm = _sc_gather(xt, tab_rm)
    out4 = _untranspose_out(hm)
    o5 = out4.reshape(NH, 8, 128, 8, 128)
    return jnp.transpose(o5, (2, 4, 0, 1, 3)).reshape(b, h, D)
